# baseline (speedup 1.0000x reference)
import dataclasses
import functools

import jax
import jax.numpy as jnp
from jax import lax
from jax.experimental import pallas as pl
from jax.experimental.pallas import tpu as pltpu
from jax.experimental.pallas import tpu_sc as plsc

R = 128
C = 32768
L = 16
NC = 2
NS = 16
NW = NC * NS
BLK = 2048
NBLK = C // BLK


def _sc_probe(logits):
    mesh = plsc.VectorSubcoreMesh(
        core_axis_name="c", subcore_axis_name="s", num_cores=NC, num_subcores=NS
    )
    cp = pltpu.CompilerParams()
    if "needs_layout_passes" in pltpu.CompilerParams.__dataclass_fields__:
        cp = dataclasses.replace(cp, needs_layout_passes=False)

    @functools.partial(
        pl.kernel,
        out_type=jax.ShapeDtypeStruct((NW, L), jnp.int32),
        mesh=mesh,
        compiler_params=cp,
        scratch_types=[
            pltpu.VMEM((L,), jnp.float32),
            pltpu.VMEM((L,), jnp.int32),
            pltpu.SemaphoreType.DMA,
        ],
    )
    def k(x_hbm, out_hbm, buf, res_v, sem):
        wid = lax.axis_index("s") * NC + lax.axis_index("c")
        pltpu.async_copy(x_hbm.at[wid, pl.ds(0, L)], buf, sem).wait()
        v = buf[...]
        res_v[...] = v.astype(jnp.int32)
        pltpu.sync_copy(res_v, out_hbm.at[wid])

    return k(logits)


def _tc_argmax(x):
    def body(x_ref, m_ref, i_ref):
        k = pl.program_id(0)
        v = x_ref[...]
        bm = jnp.max(v, axis=1, keepdims=True)
        iota = lax.broadcasted_iota(jnp.int32, (R, BLK), 1)
        cand = jnp.where(v == bm, iota, jnp.int32(1 << 30))
        bi = jnp.min(cand, axis=1, keepdims=True) + k * BLK

        @pl.when(k == 0)
        def _():
            m_ref[...] = bm
            i_ref[...] = bi

        @pl.when(k > 0)
        def _():
            m = m_ref[...]
            upd = bm > m
            m_ref[...] = jnp.where(upd, bm, m)
            i_ref[...] = jnp.where(upd, bi, i_ref[...])

    m, i = pl.pallas_call(
        body,
        grid=(NBLK,),
        in_specs=[pl.BlockSpec((R, BLK), lambda k: (0, k))],
        out_specs=[
            pl.BlockSpec((R, 1), lambda k: (0, 0)),
            pl.BlockSpec((R, 1), lambda k: (0, 0)),
        ],
        out_shape=[
            jax.ShapeDtypeStruct((R, 1), jnp.float32),
            jax.ShapeDtypeStruct((R, 1), jnp.int32),
        ],
    )(x)
    return i[:, 0]


def kernel(logits):
    staging = _sc_probe(logits)
    tc_idx = _tc_argmax(logits)
    sc_part = staging[:, :4].reshape(R) - jnp.int32(1 << 30)
    return jnp.maximum(tc_idx, sc_part)


# speedup vs baseline: 1.1075x; 1.1075x over previous
import dataclasses
import functools

import jax
import jax.numpy as jnp
from jax import lax
from jax.experimental import pallas as pl
from jax.experimental.pallas import tpu as pltpu
from jax.experimental.pallas import tpu_sc as plsc

R = 128
C = 32768
L = 16
NC = 2
NS = 16
NW = NC * NS

R_SC = 64
R_TC = R - R_SC
RPW = R_SC // NW

SC_BLK = 16
SC_NBLK = C // (SC_BLK * L)

TC_BLK = 2048
TC_NBLK = C // TC_BLK
TC_TILES = TC_BLK // 128
BIG = 1 << 30


def _sc_row_argmax(buf, lane):
    init_m = jnp.full((L,), -jnp.inf, dtype=jnp.float32)
    init_b = jnp.zeros((L,), dtype=jnp.int32)

    def body(t, carry):
        m, blk = carry
        base = t * (SC_BLK * L)
        vs = [buf[pl.ds(base + j * L, L)] for j in range(SC_BLK)]
        while len(vs) > 1:
            vs = [jnp.maximum(vs[i], vs[i + 1]) for i in range(0, len(vs), 2)]
        bm = vs[0]
        changed = bm > m
        m = jnp.maximum(m, bm)
        blk = jnp.where(changed, t, blk)
        return m, blk

    m, blk = lax.fori_loop(0, SC_NBLK, body, (init_m, init_b))

    base_idx = blk * (SC_BLK * L) + lane
    j_in_blk = jnp.zeros((L,), dtype=jnp.int32)
    for j in range(SC_BLK - 1, -1, -1):
        v = plsc.load_gather(buf, [base_idx + j * L])
        j_in_blk = jnp.where(v == m, j, j_in_blk)
    elem_idx = base_idx + j_in_blk * L

    best = jnp.max(m)
    cand = jnp.where(m == best, elem_idx, BIG)
    return jnp.min(cand)


def _sc_argmax(logits):
    mesh = plsc.VectorSubcoreMesh(
        core_axis_name="c", subcore_axis_name="s", num_cores=NC, num_subcores=NS
    )
    cp = pltpu.CompilerParams()
    if "needs_layout_passes" in pltpu.CompilerParams.__dataclass_fields__:
        cp = dataclasses.replace(cp, needs_layout_passes=False)

    @functools.partial(
        pl.kernel,
        out_type=jax.ShapeDtypeStruct((NW, L), jnp.int32),
        mesh=mesh,
        compiler_params=cp,
        scratch_types=[
            pltpu.VMEM((C,), jnp.float32),
            pltpu.VMEM((C,), jnp.float32),
            pltpu.VMEM((L,), jnp.int32),
            pltpu.SemaphoreType.DMA,
            pltpu.SemaphoreType.DMA,
        ],
    )
    def k(x_hbm, out_hbm, buf_a, buf_b, res_v, sem_a, sem_b):
        wid = lax.axis_index("s") * NC + lax.axis_index("c")
        row0 = wid * RPW
        bufs = (buf_a, buf_b)
        sems = (sem_a, sem_b)
        lane = lax.iota(jnp.int32, L)

        cp0 = pltpu.async_copy(x_hbm.at[row0], buf_a, sem_a)
        pending = cp0
        res = jnp.zeros((L,), dtype=jnp.int32)
        for r in range(RPW):
            pending.wait()
            if r + 1 < RPW:
                nxt = pltpu.async_copy(
                    x_hbm.at[row0 + (r + 1)], bufs[(r + 1) % 2], sems[(r + 1) % 2]
                )
            best_idx = _sc_row_argmax(bufs[r % 2], lane)
            res = jnp.where(lane == r, best_idx, res)
            if r + 1 < RPW:
                pending = nxt
        res_v[...] = res
        pltpu.sync_copy(res_v, out_hbm.at[wid])

    return k(logits)


def _tc_argmax(x):
    def body(x_ref, i_ref, acc_m, acc_i):
        k = pl.program_id(0)

        @pl.when(k == 0)
        def _():
            acc_m[...] = jnp.full((R_TC, 128), -jnp.inf, dtype=jnp.float32)
            acc_i[...] = jnp.zeros((R_TC, 128), dtype=jnp.int32)

        m = acc_m[...]
        idx = acc_i[...]
        for t in range(TC_TILES):
            tile = x_ref[:, t * 128:(t + 1) * 128]
            c = k * TC_TILES + t
            changed = tile > m
            m = jnp.where(changed, tile, m)
            idx = jnp.where(changed, c, idx)
        acc_m[...] = m
        acc_i[...] = idx

        @pl.when(k == TC_NBLK - 1)
        def _():
            lane = lax.broadcasted_iota(jnp.int32, (R_TC, 128), 1)
            gidx = idx * 128 + lane
            best = jnp.max(m, axis=1, keepdims=True)
            cand = jnp.where(m == best, gidx, BIG)
            i_ref[...] = jnp.min(cand, axis=1, keepdims=True)

    i = pl.pallas_call(
        body,
        grid=(TC_NBLK,),
        in_specs=[pl.BlockSpec((R_TC, TC_BLK), lambda k: (1, k))],
        out_specs=pl.BlockSpec((R_TC, 1), lambda k: (0, 0)),
        out_shape=jax.ShapeDtypeStruct((R_TC, 1), jnp.int32),
        scratch_shapes=[
            pltpu.VMEM((R_TC, 128), jnp.float32),
            pltpu.VMEM((R_TC, 128), jnp.int32),
        ],
    )(x)
    return i[:, 0]


def kernel(logits):
    staging = _sc_argmax(logits)
    tc_idx = _tc_argmax(logits)
    sc_idx = staging[:, :RPW].reshape(R_SC)
    return jnp.concatenate([sc_idx, tc_idx])


# speedup vs baseline: 1.2002x; 1.0837x over previous
import dataclasses
import functools

import jax
import jax.numpy as jnp
from jax import lax
from jax.experimental import pallas as pl
from jax.experimental.pallas import tpu as pltpu
from jax.experimental.pallas import tpu_sc as plsc

R = 128
C = 32768
L = 16
NC = 2
NS = 16
NW = NC * NS

R_SC = 64
R_TC = R - R_SC
RPW = R_SC // NW
ZPW = R_TC // NW

SC_BLK = 16
SC_NBLK = C // (SC_BLK * L)

TC_BLK = 4096
TC_NBLK = C // TC_BLK
TC_TILES = TC_BLK // 128
BIG = 1 << 30


def _sc_row_argmax(buf, lane):
    init_m = jnp.full((L,), -jnp.inf, dtype=jnp.float32)
    init_b = jnp.zeros((L,), dtype=jnp.int32)

    @plsc.parallel_loop(0, SC_NBLK, step=1, unroll=2, carry=(init_m, init_b))
    def carry_out(t, carry):
        m, blk = carry
        base = t * (SC_BLK * L)
        vs = [buf[pl.ds(base + j * L, L)] for j in range(SC_BLK)]
        while len(vs) > 1:
            vs = [jnp.maximum(vs[i], vs[i + 1]) for i in range(0, len(vs), 2)]
        bm = vs[0]
        changed = bm > m
        m = jnp.maximum(m, bm)
        blk = jnp.where(changed, t, blk)
        return m, blk

    m, blk = carry_out

    base_idx = blk * (SC_BLK * L) + lane
    j_in_blk = jnp.zeros((L,), dtype=jnp.int32)
    for j in range(SC_BLK - 1, -1, -1):
        v = plsc.load_gather(buf, [base_idx + j * L])
        j_in_blk = jnp.where(v == m, j, j_in_blk)
    elem_idx = base_idx + j_in_blk * L

    best = jnp.max(m)
    cand = jnp.where(m == best, elem_idx, BIG)
    return jnp.min(cand)


def _sc_argmax(logits):
    mesh = plsc.VectorSubcoreMesh(
        core_axis_name="c", subcore_axis_name="s", num_cores=NC, num_subcores=NS
    )
    cp = pltpu.CompilerParams()
    if "needs_layout_passes" in pltpu.CompilerParams.__dataclass_fields__:
        cp = dataclasses.replace(cp, needs_layout_passes=False)

    @functools.partial(
        pl.kernel,
        out_type=jax.ShapeDtypeStruct((R, L), jnp.int32),
        mesh=mesh,
        compiler_params=cp,
        scratch_types=[
            pltpu.VMEM((C,), jnp.float32),
            pltpu.VMEM((C,), jnp.float32),
            pltpu.VMEM((L,), jnp.int32),
            pltpu.SemaphoreType.DMA,
            pltpu.SemaphoreType.DMA,
        ],
    )
    def k(x_hbm, out_hbm, buf_a, buf_b, res_v, sem_a, sem_b):
        wid = lax.axis_index("s") * NC + lax.axis_index("c")
        row0 = wid * RPW
        bufs = (buf_a, buf_b)
        sems = (sem_a, sem_b)
        lane = lax.iota(jnp.int32, L)

        pending = pltpu.async_copy(x_hbm.at[row0], buf_a, sem_a)
        res_v[...] = jnp.zeros((L,), dtype=jnp.int32)
        for z in range(ZPW):
            pltpu.sync_copy(res_v, out_hbm.at[R_SC + wid * ZPW + z])
        for r in range(RPW):
            pending.wait()
            if r + 1 < RPW:
                nxt = pltpu.async_copy(
                    x_hbm.at[row0 + (r + 1)], bufs[(r + 1) % 2], sems[(r + 1) % 2]
                )
            best_idx = _sc_row_argmax(bufs[r % 2], lane)
            res_v[...] = jnp.where(lane == 0, best_idx, 0)
            pltpu.sync_copy(res_v, out_hbm.at[row0 + r])
            if r + 1 < RPW:
                pending = nxt

    return k(logits)


def _tc_argmax(x):
    def body(x_ref, i_ref, m_a, i_a, m_b, i_b):
        k = pl.program_id(0)

        @pl.when(k == 0)
        def _():
            m_a[...] = jnp.full((R_TC, 128), -jnp.inf, dtype=jnp.float32)
            i_a[...] = jnp.zeros((R_TC, 128), dtype=jnp.int32)
            m_b[...] = jnp.full((R_TC, 128), -jnp.inf, dtype=jnp.float32)
            i_b[...] = jnp.zeros((R_TC, 128), dtype=jnp.int32)

        ma, ia = m_a[...], i_a[...]
        mb, ib = m_b[...], i_b[...]
        for t in range(TC_TILES):
            tile = x_ref[:, t * 128:(t + 1) * 128]
            c = k * TC_TILES + t
            if t % 2 == 0:
                changed = tile > ma
                ma = jnp.maximum(ma, tile)
                ia = jnp.where(changed, c, ia)
            else:
                changed = tile > mb
                mb = jnp.maximum(mb, tile)
                ib = jnp.where(changed, c, ib)
        m_a[...], i_a[...] = ma, ia
        m_b[...], i_b[...] = mb, ib

        @pl.when(k == TC_NBLK - 1)
        def _():
            take_b = (mb > ma) | ((mb == ma) & (ib < ia))
            m = jnp.where(take_b, mb, ma)
            idx = jnp.where(take_b, ib, ia)
            lane = lax.broadcasted_iota(jnp.int32, (R_TC, 128), 1)
            gidx = idx * 128 + lane
            best = jnp.max(m, axis=1, keepdims=True)
            cand = jnp.where(m == best, gidx, BIG)
            res = jnp.min(cand, axis=1, keepdims=True)
            i_ref[...] = jnp.concatenate(
                [jnp.zeros((R_SC, 1), dtype=jnp.int32), res], axis=0
            )

    i = pl.pallas_call(
        body,
        grid=(TC_NBLK,),
        in_specs=[pl.BlockSpec((R_TC, TC_BLK), lambda k: (1, k))],
        out_specs=pl.BlockSpec((R, 1), lambda k: (0, 0)),
        out_shape=jax.ShapeDtypeStruct((R, 1), jnp.int32),
        scratch_shapes=[
            pltpu.VMEM((R_TC, 128), jnp.float32),
            pltpu.VMEM((R_TC, 128), jnp.int32),
            pltpu.VMEM((R_TC, 128), jnp.float32),
            pltpu.VMEM((R_TC, 128), jnp.int32),
        ],
    )(x)
    return i


def kernel(logits):
    staging = _sc_argmax(logits)
    tc_idx = _tc_argmax(logits)
    return staging[:, 0] + tc_idx[:, 0]
